# baseline (speedup 1.0000x reference)
import functools

import jax
import jax.numpy as jnp
from jax import lax
from jax.experimental import pallas as pl
from jax.experimental.pallas import tpu as pltpu
from jax.experimental.pallas import tpu_sc as plsc

ND = 8
BATCH = 4096
DIM = 512
EPS = 1e-3
NC = 2
NS = 16
NW = NC * NS
RPW = BATCH // NW
L = 16
CHUNKS = DIM // L
TAB = ND * DIM
SLICE = TAB // NS
CTAB = ND * L

_f32 = jnp.float32
_i32 = jnp.int32


def _mesh():
    return plsc.VectorSubcoreMesh(
        core_axis_name="c", subcore_axis_name="s",
        num_cores=NC, num_subcores=NS)


def _rsqrt(v):
    i = plsc.bitcast(v, _i32)
    i = jnp.int32(0x5F3759DF) - (i >> 1)
    y = plsc.bitcast(i, _f32)
    for _ in range(3):
        y = y * (1.5 - 0.5 * v * y * y)
    return y


def _row_domain(di_v, r):
    grp = (r // L) * L
    lane = r - grp
    chunk = di_v[pl.ds(grp, L)]
    iota = lax.iota(_i32, L)
    return jnp.max(jnp.where(iota == lane, chunk, 0))


def _stats_body(x_hbm, di_hbm, psum_hbm, pss_hbm, pcnt_hbm,
                x_v, di_v, sum_v, ss_v, cnt_v, buf_v, red_v,
                sh_sum, sh_ss, sh_cnt):
    cid = lax.axis_index("c")
    sid = lax.axis_index("s")
    wid = sid * NC + cid
    base = wid * RPW

    pltpu.sync_copy(x_hbm.at[pl.ds(base * DIM, RPW * DIM)], x_v)
    pltpu.sync_copy(di_hbm.at[pl.ds(base, RPW)], di_v)

    zeros = jnp.zeros((L,), _f32)
    ones = jnp.ones((L,), _f32)

    for c in range(TAB // L):
        sum_v[pl.ds(c * L, L)] = zeros
        ss_v[pl.ds(c * L, L)] = zeros
    for c in range(CTAB // L):
        cnt_v[pl.ds(c * L, L)] = zeros

    def row_body(r, carry):
        dom = _row_domain(di_v, r)
        tbase = dom * DIM
        for c in range(CHUNKS):
            xv = x_v[pl.ds(r * DIM + c * L, L)]
            plsc.addupdate(sum_v.at[pl.ds(tbase + c * L, L)], xv)
            plsc.addupdate(ss_v.at[pl.ds(tbase + c * L, L)], xv * xv)
        plsc.addupdate(cnt_v.at[pl.ds(dom * L, L)], ones)
        return carry

    lax.fori_loop(0, RPW, row_body, 0)

    pltpu.sync_copy(sum_v, sh_sum.at[sid])
    pltpu.sync_copy(ss_v, sh_ss.at[sid])
    pltpu.sync_copy(cnt_v, sh_cnt.at[sid])

    plsc.subcore_barrier()

    off = sid * SLICE
    for j in range(NS):
        pltpu.sync_copy(sh_sum.at[j, pl.ds(off, SLICE)], buf_v.at[j])
    for c in range(SLICE // L):
        acc = buf_v[0, pl.ds(c * L, L)]
        for j in range(1, NS):
            acc = acc + buf_v[j, pl.ds(c * L, L)]
        red_v[pl.ds(c * L, L)] = acc
    pltpu.sync_copy(red_v, psum_hbm.at[cid, pl.ds(off, SLICE)])

    for j in range(NS):
        pltpu.sync_copy(sh_ss.at[j, pl.ds(off, SLICE)], buf_v.at[j])
    for c in range(SLICE // L):
        acc = buf_v[0, pl.ds(c * L, L)]
        for j in range(1, NS):
            acc = acc + buf_v[j, pl.ds(c * L, L)]
        red_v[pl.ds(c * L, L)] = acc
    pltpu.sync_copy(red_v, pss_hbm.at[cid, pl.ds(off, SLICE)])

    @pl.when(sid == 0)
    def _():
        for j in range(NS):
            pltpu.sync_copy(sh_cnt.at[j], buf_v.at[j, pl.ds(0, CTAB)])
        for c in range(CTAB // L):
            acc = buf_v[0, pl.ds(c * L, L)]
            for j in range(1, NS):
                acc = acc + buf_v[j, pl.ds(c * L, L)]
            red_v[pl.ds(c * L, L)] = acc
        pltpu.sync_copy(red_v.at[pl.ds(0, CTAB)], pcnt_hbm.at[cid])


def _apply_body(x_hbm, di_hbm, psum_hbm, pss_hbm, pcnt_hbm,
                gg_hbm, gb_hbm, dg_hbm, db_hbm, out_hbm,
                x_v, di_v, ps_v, pq_v, pc_v,
                gg_v, gb_v, dg_v, db_v, scale_v, bias_v):
    cid = lax.axis_index("c")
    sid = lax.axis_index("s")
    wid = sid * NC + cid
    base = wid * RPW

    pltpu.sync_copy(x_hbm.at[pl.ds(base * DIM, RPW * DIM)], x_v)
    pltpu.sync_copy(di_hbm.at[pl.ds(base, RPW)], di_v)
    pltpu.sync_copy(psum_hbm, ps_v)
    pltpu.sync_copy(pss_hbm, pq_v)
    pltpu.sync_copy(pcnt_hbm, pc_v)
    pltpu.sync_copy(gg_hbm, gg_v)
    pltpu.sync_copy(gb_hbm, gb_v)
    pltpu.sync_copy(dg_hbm, dg_v)
    pltpu.sync_copy(db_hbm, db_v)

    for d in range(ND):
        cn = pc_v[pl.ds(d * L, L)] + pc_v[pl.ds(CTAB + d * L, L)]
        rc = 1.0 / jnp.maximum(cn, 1.0)
        for c in range(CHUNKS):
            o = d * DIM + c * L
            s = ps_v[pl.ds(o, L)] + ps_v[pl.ds(TAB + o, L)]
            q = pq_v[pl.ds(o, L)] + pq_v[pl.ds(TAB + o, L)]
            mean = s * rc
            var = q * rc - mean * mean + EPS
            y = _rsqrt(var)
            sc = (gg_v[pl.ds(c * L, L)] + dg_v[pl.ds(o, L)]) * y
            scale_v[pl.ds(o, L)] = sc
            bias_v[pl.ds(o, L)] = (gb_v[pl.ds(c * L, L)]
                                   + db_v[pl.ds(o, L)] - mean * sc)

    def row_body(r, carry):
        dom = _row_domain(di_v, r)
        tbase = dom * DIM
        for c in range(CHUNKS):
            off = pl.ds(r * DIM + c * L, L)
            x_v[off] = (x_v[off] * scale_v[pl.ds(tbase + c * L, L)]
                        + bias_v[pl.ds(tbase + c * L, L)])
        return carry

    lax.fori_loop(0, RPW, row_body, 0)
    pltpu.sync_copy(x_v, out_hbm.at[pl.ds(base * DIM, RPW * DIM)])


@functools.partial(pl.kernel,
                   out_type=[
                       jax.ShapeDtypeStruct((NC, TAB), _f32),
                       jax.ShapeDtypeStruct((NC, TAB), _f32),
                       jax.ShapeDtypeStruct((NC, CTAB), _f32),
                   ],
                   mesh=_mesh(),
                   compiler_params=pltpu.CompilerParams(
                       needs_layout_passes=False),
                   scratch_types=[
                       pltpu.VMEM((RPW * DIM,), _f32),
                       pltpu.VMEM((RPW,), _i32),
                       pltpu.VMEM((TAB,), _f32),
                       pltpu.VMEM((TAB,), _f32),
                       pltpu.VMEM((CTAB,), _f32),
                       pltpu.VMEM((NS, SLICE), _f32),
                       pltpu.VMEM((SLICE,), _f32),
                       pltpu.VMEM_SHARED((NS, TAB), _f32),
                       pltpu.VMEM_SHARED((NS, TAB), _f32),
                       pltpu.VMEM_SHARED((NS, CTAB), _f32),
                   ])
def _stats_call(x_hbm, di_hbm, psum_hbm, pss_hbm, pcnt_hbm, *scratch):
    _stats_body(x_hbm, di_hbm, psum_hbm, pss_hbm, pcnt_hbm, *scratch)


@functools.partial(pl.kernel,
                   out_type=jax.ShapeDtypeStruct((BATCH * DIM,), _f32),
                   mesh=_mesh(),
                   compiler_params=pltpu.CompilerParams(
                       needs_layout_passes=False),
                   scratch_types=[
                       pltpu.VMEM((RPW * DIM,), _f32),
                       pltpu.VMEM((RPW,), _i32),
                       pltpu.VMEM((NC * TAB,), _f32),
                       pltpu.VMEM((NC * TAB,), _f32),
                       pltpu.VMEM((NC * CTAB,), _f32),
                       pltpu.VMEM((DIM,), _f32),
                       pltpu.VMEM((DIM,), _f32),
                       pltpu.VMEM((TAB,), _f32),
                       pltpu.VMEM((TAB,), _f32),
                       pltpu.VMEM((TAB,), _f32),
                       pltpu.VMEM((TAB,), _f32),
                   ])
def _apply_call(x_hbm, di_hbm, psum_hbm, pss_hbm, pcnt_hbm,
                gg_hbm, gb_hbm, dg_hbm, db_hbm, out_hbm, *scratch):
    _apply_body(x_hbm, di_hbm, psum_hbm, pss_hbm, pcnt_hbm,
                gg_hbm, gb_hbm, dg_hbm, db_hbm, out_hbm, *scratch)


@jax.jit
def kernel(inputs, global_gamma, global_beta, domain_gamma, domain_beta,
           domain_index):
    x1 = inputs.reshape(-1)
    di = domain_index.astype(_i32)
    psum, pss, pcnt = _stats_call(x1, di)
    out = _apply_call(x1.reshape(-1), di,
                      psum.reshape(-1), pss.reshape(-1), pcnt.reshape(-1),
                      global_gamma, global_beta,
                      domain_gamma.reshape(-1), domain_beta.reshape(-1))
    return out.reshape(BATCH, DIM)


# speedup vs baseline: 1.2410x; 1.2410x over previous
import functools

import jax
import jax.numpy as jnp
from jax import lax
from jax.experimental import pallas as pl
from jax.experimental.pallas import tpu as pltpu
from jax.experimental.pallas import tpu_sc as plsc

ND = 8
BATCH = 4096
DIM = 512
EPS = 1e-3
NC = 2
NS = 16
NW = NC * NS
RPW = BATCH // NW
L = 16
CHUNKS = DIM // L
TAB = ND * DIM
SLICE = TAB // NS
CTAB = ND * L

_f32 = jnp.float32
_i32 = jnp.int32


def _mesh():
    return plsc.VectorSubcoreMesh(
        core_axis_name="c", subcore_axis_name="s",
        num_cores=NC, num_subcores=NS)


def _rsqrt(v):
    i = plsc.bitcast(v, _i32)
    i = jnp.int32(0x5F3759DF) - (i >> 1)
    y = plsc.bitcast(i, _f32)
    for _ in range(3):
        y = y * (1.5 - 0.5 * v * y * y)
    return y


def _row_domain(di_v, r):
    return di_v[pl.ds(r, L)][0]


def _stats_body(x_hbm, di_hbm, psum_hbm, pss_hbm, pcnt_hbm,
                x_v, di_v, sum_v, ss_v, cnt_v, buf_v, red_v,
                sh_sum, sh_ss, sh_cnt):
    cid = lax.axis_index("c")
    sid = lax.axis_index("s")
    wid = sid * NC + cid
    base = wid * RPW

    pltpu.sync_copy(x_hbm.at[pl.ds(base * DIM, RPW * DIM)], x_v)
    pltpu.sync_copy(di_hbm.at[pl.ds(base, RPW)], di_v.at[pl.ds(0, RPW)])

    zeros = jnp.zeros((L,), _f32)
    ones = jnp.ones((L,), _f32)

    for c in range(TAB // L):
        sum_v[pl.ds(c * L, L)] = zeros
        ss_v[pl.ds(c * L, L)] = zeros
    for c in range(CTAB // L):
        cnt_v[pl.ds(c * L, L)] = zeros

    def row_body(r, dom):
        dom_next = _row_domain(di_v, r + 1)
        tbase = dom * DIM
        rbase = r * DIM
        xv = x_v[pl.ds(rbase, L)]
        for c in range(CHUNKS):
            if c + 1 < CHUNKS:
                xn = x_v[pl.ds(rbase + (c + 1) * L, L)]
            plsc.addupdate(sum_v.at[pl.ds(tbase + c * L, L)], xv)
            plsc.addupdate(ss_v.at[pl.ds(tbase + c * L, L)], xv * xv)
            if c + 1 < CHUNKS:
                xv = xn
        plsc.addupdate(cnt_v.at[pl.ds(dom * L, L)], ones)
        return dom_next

    lax.fori_loop(0, RPW, row_body, _row_domain(di_v, 0))

    pltpu.sync_copy(sum_v, sh_sum.at[sid])
    pltpu.sync_copy(ss_v, sh_ss.at[sid])
    pltpu.sync_copy(cnt_v, sh_cnt.at[sid])

    plsc.subcore_barrier()

    off = sid * SLICE
    for j in range(NS):
        pltpu.sync_copy(sh_sum.at[j, pl.ds(off, SLICE)], buf_v.at[j])
    for c in range(SLICE // L):
        acc = buf_v[0, pl.ds(c * L, L)]
        for j in range(1, NS):
            acc = acc + buf_v[j, pl.ds(c * L, L)]
        red_v[pl.ds(c * L, L)] = acc
    pltpu.sync_copy(red_v, psum_hbm.at[cid, pl.ds(off, SLICE)])

    for j in range(NS):
        pltpu.sync_copy(sh_ss.at[j, pl.ds(off, SLICE)], buf_v.at[j])
    for c in range(SLICE // L):
        acc = buf_v[0, pl.ds(c * L, L)]
        for j in range(1, NS):
            acc = acc + buf_v[j, pl.ds(c * L, L)]
        red_v[pl.ds(c * L, L)] = acc
    pltpu.sync_copy(red_v, pss_hbm.at[cid, pl.ds(off, SLICE)])

    @pl.when(sid == 0)
    def _():
        for j in range(NS):
            pltpu.sync_copy(sh_cnt.at[j], buf_v.at[j, pl.ds(0, CTAB)])
        for c in range(CTAB // L):
            acc = buf_v[0, pl.ds(c * L, L)]
            for j in range(1, NS):
                acc = acc + buf_v[j, pl.ds(c * L, L)]
            red_v[pl.ds(c * L, L)] = acc
        pltpu.sync_copy(red_v.at[pl.ds(0, CTAB)], pcnt_hbm.at[cid])


def _apply_body(x_hbm, di_hbm, psum_hbm, pss_hbm, pcnt_hbm,
                gg_hbm, gb_hbm, dg_hbm, db_hbm, out_hbm,
                x_v, di_v, ps_v, pq_v, pc_v,
                gg_v, gb_v, dg_v, db_v, scale_v, bias_v):
    cid = lax.axis_index("c")
    sid = lax.axis_index("s")
    wid = sid * NC + cid
    base = wid * RPW

    pltpu.sync_copy(x_hbm.at[pl.ds(base * DIM, RPW * DIM)], x_v)
    pltpu.sync_copy(di_hbm.at[pl.ds(base, RPW)], di_v.at[pl.ds(0, RPW)])
    pltpu.sync_copy(psum_hbm, ps_v)
    pltpu.sync_copy(pss_hbm, pq_v)
    pltpu.sync_copy(pcnt_hbm, pc_v)
    pltpu.sync_copy(gg_hbm, gg_v)
    pltpu.sync_copy(gb_hbm, gb_v)
    pltpu.sync_copy(dg_hbm, dg_v)
    pltpu.sync_copy(db_hbm, db_v)

    for d in range(ND):
        cn = pc_v[pl.ds(d * L, L)] + pc_v[pl.ds(CTAB + d * L, L)]
        rc = 1.0 / jnp.maximum(cn, 1.0)
        for c in range(CHUNKS):
            o = d * DIM + c * L
            s = ps_v[pl.ds(o, L)] + ps_v[pl.ds(TAB + o, L)]
            q = pq_v[pl.ds(o, L)] + pq_v[pl.ds(TAB + o, L)]
            mean = s * rc
            var = q * rc - mean * mean + EPS
            y = _rsqrt(var)
            sc = (gg_v[pl.ds(c * L, L)] + dg_v[pl.ds(o, L)]) * y
            scale_v[pl.ds(o, L)] = sc
            bias_v[pl.ds(o, L)] = (gb_v[pl.ds(c * L, L)]
                                   + db_v[pl.ds(o, L)] - mean * sc)

    def row_body(r, dom):
        dom_next = _row_domain(di_v, r + 1)
        tbase = dom * DIM
        rbase = r * DIM
        xv = x_v[pl.ds(rbase, L)]
        sv = scale_v[pl.ds(tbase, L)]
        bv = bias_v[pl.ds(tbase, L)]
        for c in range(CHUNKS):
            if c + 1 < CHUNKS:
                xn = x_v[pl.ds(rbase + (c + 1) * L, L)]
                sn = scale_v[pl.ds(tbase + (c + 1) * L, L)]
                bn = bias_v[pl.ds(tbase + (c + 1) * L, L)]
            x_v[pl.ds(rbase + c * L, L)] = xv * sv + bv
            if c + 1 < CHUNKS:
                xv, sv, bv = xn, sn, bn
        return dom_next

    lax.fori_loop(0, RPW, row_body, _row_domain(di_v, 0))
    pltpu.sync_copy(x_v, out_hbm.at[pl.ds(base * DIM, RPW * DIM)])


@functools.partial(pl.kernel,
                   out_type=[
                       jax.ShapeDtypeStruct((NC, TAB), _f32),
                       jax.ShapeDtypeStruct((NC, TAB), _f32),
                       jax.ShapeDtypeStruct((NC, CTAB), _f32),
                   ],
                   mesh=_mesh(),
                   compiler_params=pltpu.CompilerParams(
                       needs_layout_passes=False),
                   scratch_types=[
                       pltpu.VMEM((RPW * DIM,), _f32),
                       pltpu.VMEM((RPW + L,), _i32),
                       pltpu.VMEM((TAB,), _f32),
                       pltpu.VMEM((TAB,), _f32),
                       pltpu.VMEM((CTAB,), _f32),
                       pltpu.VMEM((NS, SLICE), _f32),
                       pltpu.VMEM((SLICE,), _f32),
                       pltpu.VMEM_SHARED((NS, TAB), _f32),
                       pltpu.VMEM_SHARED((NS, TAB), _f32),
                       pltpu.VMEM_SHARED((NS, CTAB), _f32),
                   ])
def _stats_call(x_hbm, di_hbm, psum_hbm, pss_hbm, pcnt_hbm, *scratch):
    _stats_body(x_hbm, di_hbm, psum_hbm, pss_hbm, pcnt_hbm, *scratch)


@functools.partial(pl.kernel,
                   out_type=jax.ShapeDtypeStruct((BATCH * DIM,), _f32),
                   mesh=_mesh(),
                   compiler_params=pltpu.CompilerParams(
                       needs_layout_passes=False),
                   scratch_types=[
                       pltpu.VMEM((RPW * DIM,), _f32),
                       pltpu.VMEM((RPW + L,), _i32),
                       pltpu.VMEM((NC * TAB,), _f32),
                       pltpu.VMEM((NC * TAB,), _f32),
                       pltpu.VMEM((NC * CTAB,), _f32),
                       pltpu.VMEM((DIM,), _f32),
                       pltpu.VMEM((DIM,), _f32),
                       pltpu.VMEM((TAB,), _f32),
                       pltpu.VMEM((TAB,), _f32),
                       pltpu.VMEM((TAB,), _f32),
                       pltpu.VMEM((TAB,), _f32),
                   ])
def _apply_call(x_hbm, di_hbm, psum_hbm, pss_hbm, pcnt_hbm,
                gg_hbm, gb_hbm, dg_hbm, db_hbm, out_hbm, *scratch):
    _apply_body(x_hbm, di_hbm, psum_hbm, pss_hbm, pcnt_hbm,
                gg_hbm, gb_hbm, dg_hbm, db_hbm, out_hbm, *scratch)


@jax.jit
def kernel(inputs, global_gamma, global_beta, domain_gamma, domain_beta,
           domain_index):
    x1 = inputs.reshape(-1)
    di = domain_index.astype(_i32)
    psum, pss, pcnt = _stats_call(x1, di)
    out = _apply_call(x1.reshape(-1), di,
                      psum.reshape(-1), pss.reshape(-1), pcnt.reshape(-1),
                      global_gamma, global_beta,
                      domain_gamma.reshape(-1), domain_beta.reshape(-1))
    return out.reshape(BATCH, DIM)


# speedup vs baseline: 2.0188x; 1.6268x over previous
import functools

import jax
import jax.numpy as jnp
from jax import lax
from jax.experimental import pallas as pl
from jax.experimental.pallas import tpu as pltpu
from jax.experimental.pallas import tpu_sc as plsc

ND = 8
BATCH = 4096
DIM = 512
EPS = 1e-3
NC = 2
NS = 16
RPW = BATCH // NS
CPW = DIM // NC
L = 16
CHUNKS = CPW // L
TAB = ND * CPW
SLICE = TAB // NS
CTAB = ND * L

_f32 = jnp.float32
_i32 = jnp.int32


def _rsqrt(v):
    i = plsc.bitcast(v, _i32)
    i = jnp.int32(0x5F3759DF) - (i >> 1)
    y = plsc.bitcast(i, _f32)
    for _ in range(3):
        y = y * (1.5 - 0.5 * v * y * y)
    return y


def _row_domain(di_v, r):
    return di_v[pl.ds(r, L)][0]


def _body(x_hbm, di_hbm, gg_hbm, gb_hbm, dg_hbm, db_hbm, out_hbm,
          x_v, di_v, sum_v, ss_v, cnt_v, buf_v,
          gg_v, gb_v, dg_v, db_v, scale_v, bias_v,
          sh_sum, sh_ss, sh_cnt, sh_csum, sh_css, sh_ccnt):
    cid = lax.axis_index("c")
    sid = lax.axis_index("s")
    r0 = sid * RPW
    c0 = cid * CPW

    pltpu.sync_copy(x_hbm.at[pl.ds(r0, RPW), pl.ds(c0, CPW)], x_v)
    pltpu.sync_copy(di_hbm.at[pl.ds(r0, RPW)], di_v.at[pl.ds(0, RPW)])
    pltpu.sync_copy(gg_hbm.at[pl.ds(c0, CPW)], gg_v)
    pltpu.sync_copy(gb_hbm.at[pl.ds(c0, CPW)], gb_v)
    pltpu.sync_copy(dg_hbm.at[:, pl.ds(c0, CPW)], dg_v)
    pltpu.sync_copy(db_hbm.at[:, pl.ds(c0, CPW)], db_v)

    zeros = jnp.zeros((L,), _f32)
    ones = jnp.ones((L,), _f32)

    for c in range(TAB // L):
        sum_v[pl.ds(c * L, L)] = zeros
        ss_v[pl.ds(c * L, L)] = zeros
    for c in range(CTAB // L):
        cnt_v[pl.ds(c * L, L)] = zeros

    def stats_row(r, dom):
        dom_next = _row_domain(di_v, r + 1)
        tbase = dom * CPW
        xv = x_v[r, pl.ds(0, L)]
        for c in range(CHUNKS):
            if c + 1 < CHUNKS:
                xn = x_v[r, pl.ds((c + 1) * L, L)]
            plsc.addupdate(sum_v.at[pl.ds(tbase + c * L, L)], xv)
            plsc.addupdate(ss_v.at[pl.ds(tbase + c * L, L)], xv * xv)
            if c + 1 < CHUNKS:
                xv = xn
        plsc.addupdate(cnt_v.at[pl.ds(dom * L, L)], ones)
        return dom_next

    lax.fori_loop(0, RPW, stats_row, _row_domain(di_v, 0))

    pltpu.sync_copy(sum_v, sh_sum.at[sid])
    pltpu.sync_copy(ss_v, sh_ss.at[sid])
    pltpu.sync_copy(cnt_v, sh_cnt.at[sid])

    plsc.subcore_barrier()

    off = sid * SLICE
    for j in range(NS):
        pltpu.sync_copy(sh_sum.at[j, pl.ds(off, SLICE)], buf_v.at[j, pl.ds(0, SLICE)])
    for c in range(SLICE // L):
        acc = buf_v[0, pl.ds(c * L, L)]
        for j in range(1, NS):
            acc = acc + buf_v[j, pl.ds(c * L, L)]
        sum_v[pl.ds(c * L, L)] = acc
    pltpu.sync_copy(sum_v.at[pl.ds(0, SLICE)], sh_csum.at[pl.ds(off, SLICE)])

    for j in range(NS):
        pltpu.sync_copy(sh_ss.at[j, pl.ds(off, SLICE)], buf_v.at[j, pl.ds(0, SLICE)])
    for c in range(SLICE // L):
        acc = buf_v[0, pl.ds(c * L, L)]
        for j in range(1, NS):
            acc = acc + buf_v[j, pl.ds(c * L, L)]
        ss_v[pl.ds(c * L, L)] = acc
    pltpu.sync_copy(ss_v.at[pl.ds(0, SLICE)], sh_css.at[pl.ds(off, SLICE)])

    @pl.when(sid == 0)
    def _():
        for j in range(NS):
            pltpu.sync_copy(sh_cnt.at[j], buf_v.at[j, pl.ds(0, CTAB)])
        for c in range(CTAB // L):
            acc = buf_v[0, pl.ds(c * L, L)]
            for j in range(1, NS):
                acc = acc + buf_v[j, pl.ds(c * L, L)]
            cnt_v[pl.ds(c * L, L)] = acc
        pltpu.sync_copy(cnt_v, sh_ccnt)

    plsc.subcore_barrier()

    pltpu.sync_copy(sh_csum, sum_v)
    pltpu.sync_copy(sh_css, ss_v)
    pltpu.sync_copy(sh_ccnt, cnt_v)

    for d in range(ND):
        cn = cnt_v[pl.ds(d * L, L)]
        rc = 1.0 / jnp.maximum(cn, 1.0)
        for c in range(CHUNKS):
            o = d * CPW + c * L
            mean = sum_v[pl.ds(o, L)] * rc
            var = ss_v[pl.ds(o, L)] * rc - mean * mean + EPS
            y = _rsqrt(var)
            sc = (gg_v[pl.ds(c * L, L)] + dg_v[d, pl.ds(c * L, L)]) * y
            scale_v[pl.ds(o, L)] = sc
            bias_v[pl.ds(o, L)] = (gb_v[pl.ds(c * L, L)]
                                   + db_v[d, pl.ds(c * L, L)] - mean * sc)

    def apply_row(r, dom):
        dom_next = _row_domain(di_v, r + 1)
        tbase = dom * CPW
        xv = x_v[r, pl.ds(0, L)]
        sv = scale_v[pl.ds(tbase, L)]
        bv = bias_v[pl.ds(tbase, L)]
        for c in range(CHUNKS):
            if c + 1 < CHUNKS:
                xn = x_v[r, pl.ds((c + 1) * L, L)]
                sn = scale_v[pl.ds(tbase + (c + 1) * L, L)]
                bn = bias_v[pl.ds(tbase + (c + 1) * L, L)]
            x_v[r, pl.ds(c * L, L)] = xv * sv + bv
            if c + 1 < CHUNKS:
                xv, sv, bv = xn, sn, bn
        return dom_next

    lax.fori_loop(0, RPW, apply_row, _row_domain(di_v, 0))
    pltpu.sync_copy(x_v, out_hbm.at[pl.ds(r0, RPW), pl.ds(c0, CPW)])


@functools.partial(pl.kernel,
                   out_type=jax.ShapeDtypeStruct((BATCH, DIM), _f32),
                   mesh=plsc.VectorSubcoreMesh(
                       core_axis_name="c", subcore_axis_name="s",
                       num_cores=NC, num_subcores=NS),
                   compiler_params=pltpu.CompilerParams(
                       needs_layout_passes=False),
                   scratch_types=[
                       pltpu.VMEM((RPW, CPW), _f32),
                       pltpu.VMEM((RPW + L,), _i32),
                       pltpu.VMEM((TAB,), _f32),
                       pltpu.VMEM((TAB,), _f32),
                       pltpu.VMEM((CTAB,), _f32),
                       pltpu.VMEM((NS, SLICE), _f32),
                       pltpu.VMEM((CPW,), _f32),
                       pltpu.VMEM((CPW,), _f32),
                       pltpu.VMEM((ND, CPW), _f32),
                       pltpu.VMEM((ND, CPW), _f32),
                       pltpu.VMEM((TAB,), _f32),
                       pltpu.VMEM((TAB,), _f32),
                       pltpu.VMEM_SHARED((NS, TAB), _f32),
                       pltpu.VMEM_SHARED((NS, TAB), _f32),
                       pltpu.VMEM_SHARED((NS, CTAB), _f32),
                       pltpu.VMEM_SHARED((TAB,), _f32),
                       pltpu.VMEM_SHARED((TAB,), _f32),
                       pltpu.VMEM_SHARED((CTAB,), _f32),
                   ])
def _pn_call(x_hbm, di_hbm, gg_hbm, gb_hbm, dg_hbm, db_hbm, out_hbm,
             *scratch):
    _body(x_hbm, di_hbm, gg_hbm, gb_hbm, dg_hbm, db_hbm, out_hbm, *scratch)


@jax.jit
def kernel(inputs, global_gamma, global_beta, domain_gamma, domain_beta,
           domain_index):
    di = domain_index.astype(_i32)
    return _pn_call(inputs, di, global_gamma, global_beta,
                    domain_gamma, domain_beta)
